# baseline (speedup 1.0000x reference)
import functools

import jax
import jax.numpy as jnp
from jax.experimental import pallas as pl
from jax.experimental.pallas import tpu as pltpu

T = 2048
D = 2048
H = 16
DQN = 128
DR = 64
DV = 128
QLR = 1536
KVLR = 512
E = 64
K = 8
F = 128
EPS = 1e-06
SCALE = (DQN + DR) ** -0.5
RSF = 2.5
CAP = int(T * K / E * 2)

BM = 256


def _rms_bf16(x, w):
    v = jnp.mean(x * x, axis=-1, keepdims=True)
    return (x * jax.lax.rsqrt(v + EPS) * w).astype(jnp.bfloat16)


def _mm_rms_body(x_ref, wln_ref, w_ref, o_ref):
    xn = _rms_bf16(x_ref[...], wln_ref[...])
    o_ref[...] = jnp.dot(xn, w_ref[...].astype(jnp.bfloat16),
                         preferred_element_type=jnp.float32)


def _mm_rms(x, wln, w, kwidth=None, colblk=0):
    t = x.shape[0]
    k = w.shape[0] if kwidth is None else kwidth
    n = w.shape[1]
    return pl.pallas_call(
        _mm_rms_body,
        grid=(t // BM,),
        in_specs=[
            pl.BlockSpec((BM, k), lambda i: (i, colblk)),
            pl.BlockSpec((1, k), lambda i: (0, 0)),
            pl.BlockSpec((k, n), lambda i: (0, 0)),
        ],
        out_specs=pl.BlockSpec((BM, n), lambda i: (i, 0)),
        out_shape=jax.ShapeDtypeStruct((t, n), jnp.float32),
    )(x, wln.reshape(1, k), w)


def _rope_i(x, c, s):
    rp = jnp.roll(x, 1, axis=1)
    rm = jnp.roll(x, -1, axis=1)
    lane = jax.lax.broadcasted_iota(jnp.int32, x.shape, 1)
    swap = jnp.where(lane % 2 == 0, -rm, rp)
    return x * c + swap * s


def _attn_body(q_ref, kpe_ref, kv_ref, cos_ref, sin_ref, o_ref):
    qb = pl.program_id(1)
    q0 = qb * BM
    cq = cos_ref[pl.ds(q0, BM), :]
    sq = sin_ref[pl.ds(q0, BM), :]
    qblk = q_ref[...]
    row = q0 + jax.lax.broadcasted_iota(jnp.int32, (BM, BM), 0)
    cloc = jax.lax.broadcasted_iota(jnp.int32, (BM, BM), 1)
    for hh in range(2):
        qh = qblk[:, hh * 192:(hh + 1) * 192]
        qf = jnp.concatenate(
            [qh[:, :DQN], _rope_i(qh[:, DQN:], cq, sq)], axis=1
        ).astype(jnp.bfloat16)

        def chunk(c, carry):
            m, l, acc = carry
            kn = kv_ref[pl.ds(c * BM, BM), hh * 256:hh * 256 + DQN]
            kpe = _rope_i(kpe_ref[pl.ds(c * BM, BM), :],
                          cos_ref[pl.ds(c * BM, BM), :],
                          sin_ref[pl.ds(c * BM, BM), :])
            kf = jnp.concatenate([kn, kpe], axis=1).astype(jnp.bfloat16)
            v = kv_ref[pl.ds(c * BM, BM),
                       hh * 256 + DQN:(hh + 1) * 256].astype(jnp.bfloat16)
            s = jax.lax.dot_general(
                qf, kf, (((1,), (1,)), ((), ())),
                preferred_element_type=jnp.float32) * SCALE
            s = jnp.where(c * BM + cloc <= row, s, -1e30)
            mc = jnp.max(s, axis=1, keepdims=True)
            mn = jnp.maximum(m, mc)
            corr = jnp.exp(m - mn)
            p = jnp.exp(s - mn)
            ln = l * corr + jnp.sum(p, axis=1, keepdims=True)
            accn = acc * corr + jnp.dot(p.astype(jnp.bfloat16), v,
                                        preferred_element_type=jnp.float32)
            return mn, ln, accn

        m0 = jnp.full((BM, 1), -jnp.inf, jnp.float32)
        l0 = jnp.zeros((BM, 1), jnp.float32)
        a0 = jnp.zeros((BM, DV), jnp.float32)
        _, l, acc = jax.lax.fori_loop(0, qb + 1, chunk, (m0, l0, a0))
        o_ref[:, hh * DV:(hh + 1) * DV] = acc / l


def _attention(q, k_pe, kv, cos2, sin2):
    return pl.pallas_call(
        _attn_body,
        grid=(H // 2, T // BM),
        in_specs=[
            pl.BlockSpec((BM, 384), lambda h, q: (q, h)),
            pl.BlockSpec((T, DR), lambda h, q: (0, 0)),
            pl.BlockSpec((T, 512), lambda h, q: (0, h)),
            pl.BlockSpec((T, DR), lambda h, q: (0, 0)),
            pl.BlockSpec((T, DR), lambda h, q: (0, 0)),
        ],
        out_specs=pl.BlockSpec((BM, 2 * DV), lambda h, q: (q, h)),
        out_shape=jax.ShapeDtypeStruct((T, H * DV), jnp.float32),
    )(q, k_pe, kv, cos2, sin2)


def _wo_body(o_ref, w_ref, hid_ref, h_ref):
    h_ref[...] = hid_ref[...] + jnp.dot(
        o_ref[...].astype(jnp.bfloat16), w_ref[...].astype(jnp.bfloat16),
        preferred_element_type=jnp.float32)


def _wo_res(o_cat, w_o, hidden):
    return pl.pallas_call(
        _wo_body,
        grid=(T // BM,),
        in_specs=[
            pl.BlockSpec((BM, H * DV), lambda i: (i, 0)),
            pl.BlockSpec((H * DV, D), lambda i: (0, 0)),
            pl.BlockSpec((BM, D), lambda i: (i, 0)),
        ],
        out_specs=pl.BlockSpec((BM, D), lambda i: (i, 0)),
        out_shape=jax.ShapeDtypeStruct((T, D), jnp.float32),
    )(o_cat, w_o, hidden)


def _router_body(h_ref, wln_ref, wr_ref, eb_ref, x2_ref, sel_ref, w_ref):
    h = h_ref[...]
    v = jnp.mean(h * h, axis=-1, keepdims=True)
    x2 = h * jax.lax.rsqrt(v + EPS) * wln_ref[...]
    x2_ref[...] = x2.astype(jnp.bfloat16)
    logits = jnp.dot(x2, wr_ref[...], preferred_element_type=jnp.float32)
    sc = jax.nn.sigmoid(logits)
    b = sc + eb_ref[...]
    idx = jax.lax.broadcasted_iota(jnp.int32, (BM, E), 1)
    sel = jnp.zeros((BM, E), jnp.float32)
    for _ in range(K):
        m = jnp.max(b, axis=1, keepdims=True)
        am = jnp.min(jnp.where(b == m, idx, E), axis=1, keepdims=True)
        pick = idx == am
        sel = jnp.where(pick, 1.0, sel)
        b = jnp.where(pick, -jnp.inf, b)
    sw = sel * sc
    denom = jnp.sum(sw, axis=1, keepdims=True) + 1e-20
    sel_ref[...] = sel
    w_ref[...] = sw / denom * RSF


def _router(h, post_ln_w, w_router, e_bias):
    return pl.pallas_call(
        _router_body,
        grid=(T // BM,),
        in_specs=[
            pl.BlockSpec((BM, D), lambda i: (i, 0)),
            pl.BlockSpec((1, D), lambda i: (0, 0)),
            pl.BlockSpec((D, E), lambda i: (0, 0)),
            pl.BlockSpec((1, E), lambda i: (0, 0)),
        ],
        out_specs=[
            pl.BlockSpec((BM, D), lambda i: (i, 0)),
            pl.BlockSpec((BM, E), lambda i: (i, 0)),
            pl.BlockSpec((BM, E), lambda i: (i, 0)),
        ],
        out_shape=[
            jax.ShapeDtypeStruct((T, D), jnp.bfloat16),
            jax.ShapeDtypeStruct((T, E), jnp.float32),
            jax.ShapeDtypeStruct((T, E), jnp.float32),
        ],
    )(h, post_ln_w.reshape(1, D), w_router, e_bias.reshape(1, E))


def _cap_body(sel_ref, w_ref, kw_ref, carry_ref):
    i = pl.program_id(0)

    @pl.when(i == 0)
    def _():
        carry_ref[...] = jnp.zeros_like(carry_ref)

    sel = sel_ref[...]
    r = jax.lax.broadcasted_iota(jnp.int32, (BM, BM), 0)
    c = jax.lax.broadcasted_iota(jnp.int32, (BM, BM), 1)
    tril = (r >= c).astype(jnp.bfloat16)
    cs = jnp.dot(tril, sel.astype(jnp.bfloat16),
                 preferred_element_type=jnp.float32)
    pie = cs - 1.0 + carry_ref[...]
    keep = jnp.where(pie < CAP, sel, 0.0)
    kw_ref[...] = w_ref[...] * keep
    carry_ref[...] += cs[BM - 1:BM, :]


def _capacity(sel, wfull):
    return pl.pallas_call(
        _cap_body,
        grid=(T // BM,),
        in_specs=[
            pl.BlockSpec((BM, E), lambda i: (i, 0)),
            pl.BlockSpec((BM, E), lambda i: (i, 0)),
        ],
        out_specs=pl.BlockSpec((BM, E), lambda i: (i, 0)),
        out_shape=jax.ShapeDtypeStruct((T, E), jnp.float32),
        scratch_shapes=[pltpu.VMEM((1, E), jnp.float32)],
    )(sel, wfull)


def _exp_body(x2_ref, wg_ref, wu_ref, kw_ref, hg_ref):
    e = pl.program_id(0)
    x2 = x2_ref[...]
    xg = jnp.dot(x2, wg_ref[0].astype(jnp.bfloat16),
                 preferred_element_type=jnp.float32)
    xu = jnp.dot(x2, wu_ref[0].astype(jnp.bfloat16),
                 preferred_element_type=jnp.float32)
    ohe = (jax.lax.broadcasted_iota(jnp.int32, (1, E), 1) == e
           ).astype(jnp.float32)
    kcol = jnp.sum(kw_ref[...] * ohe, axis=1, keepdims=True)
    hg = jax.nn.silu(xg) * xu * kcol
    hg_ref[...] = hg.astype(jnp.bfloat16)


def _experts_hg(x2b, w_g, w_u, keepw):
    return pl.pallas_call(
        _exp_body,
        grid=(E,),
        in_specs=[
            pl.BlockSpec((T, D), lambda e: (0, 0)),
            pl.BlockSpec((1, D, F), lambda e: (e, 0, 0)),
            pl.BlockSpec((1, D, F), lambda e: (e, 0, 0)),
            pl.BlockSpec((T, E), lambda e: (0, 0)),
        ],
        out_specs=pl.BlockSpec((T, F), lambda e: (0, e)),
        out_shape=jax.ShapeDtypeStruct((T, E * F), jnp.bfloat16),
    )(x2b, w_g, w_u, keepw)


def _final_body(hg_ref, wd_ref, h_ref, x2_ref, wsg_ref, wsu_ref, wsd_ref,
                o_ref):
    x2 = x2_ref[...]
    sg = jnp.dot(x2, wsg_ref[...].astype(jnp.bfloat16),
                 preferred_element_type=jnp.float32)
    su = jnp.dot(x2, wsu_ref[...].astype(jnp.bfloat16),
                 preferred_element_type=jnp.float32)
    hs = (jax.nn.silu(sg) * su).astype(jnp.bfloat16)
    shared = jnp.dot(hs, wsd_ref[...].astype(jnp.bfloat16),
                     preferred_element_type=jnp.float32)
    routed = jnp.dot(hg_ref[...], wd_ref[...],
                     preferred_element_type=jnp.float32)
    o_ref[...] = h_ref[...] + routed + shared


def _final(hg, wd_stack_bf16, h, x2b, ws_g, ws_u, ws_d):
    return pl.pallas_call(
        _final_body,
        grid=(T // BM,),
        in_specs=[
            pl.BlockSpec((BM, E * F), lambda i: (i, 0)),
            pl.BlockSpec((E * F, D), lambda i: (0, 0)),
            pl.BlockSpec((BM, D), lambda i: (i, 0)),
            pl.BlockSpec((BM, D), lambda i: (i, 0)),
            pl.BlockSpec((D, F), lambda i: (0, 0)),
            pl.BlockSpec((D, F), lambda i: (0, 0)),
            pl.BlockSpec((F, D), lambda i: (0, 0)),
        ],
        out_specs=pl.BlockSpec((BM, D), lambda i: (i, 0)),
        out_shape=jax.ShapeDtypeStruct((T, D), jnp.float32),
    )(hg, wd_stack_bf16, h, x2b, ws_g, ws_u, ws_d)


def kernel(hidden_states, positions, input_ln_w, post_ln_w, W_qkv_a,
           q_a_ln_w, W_q_b, kv_a_ln_w, W_kv_b, W_o, W_router, e_bias,
           W_g, W_u, W_d, Ws_g, Ws_u, Ws_d):
    inv = 1.0 / (10000.0 ** (jnp.arange(0, DR, 2, dtype=jnp.float32) / DR))
    ang = positions.astype(jnp.float32)[:, None] * inv[None, :]
    cos2 = jnp.repeat(jnp.cos(ang), 2, axis=1)
    sin2 = jnp.repeat(jnp.sin(ang), 2, axis=1)

    qkv = _mm_rms(hidden_states, input_ln_w, W_qkv_a)
    q = _mm_rms(qkv, q_a_ln_w, W_q_b, kwidth=QLR, colblk=0)
    kv = _mm_rms(qkv, kv_a_ln_w, W_kv_b, kwidth=KVLR, colblk=QLR // KVLR)

    o_cat = _attention(q, qkv[:, QLR + KVLR:], kv, cos2, sin2)
    h = _wo_res(o_cat, W_o, hidden_states)

    x2b, sel, wfull = _router(h, post_ln_w, W_router, e_bias)
    keepw = _capacity(sel, wfull)

    hg = _experts_hg(x2b, W_g, W_u, keepw)
    wd_stack = W_d.reshape(E * F, D).astype(jnp.bfloat16)
    out = _final(hg, wd_stack, h, x2b, Ws_g, Ws_u, Ws_d)
    return out


# speedup vs baseline: 1.3104x; 1.3104x over previous
import functools

import jax
import jax.numpy as jnp
from jax.experimental import pallas as pl
from jax.experimental.pallas import tpu as pltpu

T = 2048
D = 2048
H = 16
DQN = 128
DR = 64
DV = 128
QLR = 1536
KVLR = 512
E = 64
K = 8
F = 128
EPS = 1e-06
SCALE = (DQN + DR) ** -0.5
RSF = 2.5
CAP = int(T * K / E * 2)

BM = 256


def _rms_bf16(x, w):
    v = jnp.mean(x * x, axis=-1, keepdims=True)
    return (x * jax.lax.rsqrt(v + EPS) * w).astype(jnp.bfloat16)


def _mm_rms_body(x_ref, wln_ref, w_ref, o_ref):
    xn = _rms_bf16(x_ref[...], wln_ref[...])
    o_ref[...] = jnp.dot(xn, w_ref[...].astype(jnp.bfloat16),
                         preferred_element_type=jnp.float32)


def _mm_rms(x, wln, w, kwidth=None, colblk=0):
    t = x.shape[0]
    k = w.shape[0] if kwidth is None else kwidth
    n = w.shape[1]
    return pl.pallas_call(
        _mm_rms_body,
        grid=(t // BM,),
        in_specs=[
            pl.BlockSpec((BM, k), lambda i: (i, colblk)),
            pl.BlockSpec((1, k), lambda i: (0, 0)),
            pl.BlockSpec((k, n), lambda i: (0, 0)),
        ],
        out_specs=pl.BlockSpec((BM, n), lambda i: (i, 0)),
        out_shape=jax.ShapeDtypeStruct((t, n), jnp.float32),
    )(x, wln.reshape(1, k), w)


def _kv_body(x_ref, wln_ref, w_ref, kpe_ref, cos_ref, sin_ref,
             kf_ref, v_ref):
    xn = _rms_bf16(x_ref[...], wln_ref[...])
    kv = jnp.dot(xn, w_ref[...].astype(jnp.bfloat16),
                 preferred_element_type=jnp.float32)
    kpe = _rope_i(kpe_ref[...], cos_ref[...], sin_ref[...])
    kfs = []
    vs = []
    for h in range(H):
        kfs.append(kv[:, h * 256:h * 256 + DQN])
        kfs.append(kpe)
        vs.append(kv[:, h * 256 + DQN:(h + 1) * 256])
    kf_ref[...] = jnp.concatenate(kfs, axis=1).astype(jnp.bfloat16)
    v_ref[...] = jnp.concatenate(vs, axis=1).astype(jnp.bfloat16)


def _kv_proj(qkv, kv_a_ln_w, W_kv_b, k_pe, cos2, sin2):
    return pl.pallas_call(
        _kv_body,
        grid=(T // BM,),
        in_specs=[
            pl.BlockSpec((BM, KVLR), lambda i: (i, QLR // KVLR)),
            pl.BlockSpec((1, KVLR), lambda i: (0, 0)),
            pl.BlockSpec((KVLR, H * 256), lambda i: (0, 0)),
            pl.BlockSpec((BM, DR), lambda i: (i, 0)),
            pl.BlockSpec((BM, DR), lambda i: (i, 0)),
            pl.BlockSpec((BM, DR), lambda i: (i, 0)),
        ],
        out_specs=[
            pl.BlockSpec((BM, H * 192), lambda i: (i, 0)),
            pl.BlockSpec((BM, H * DV), lambda i: (i, 0)),
        ],
        out_shape=[
            jax.ShapeDtypeStruct((T, H * 192), jnp.bfloat16),
            jax.ShapeDtypeStruct((T, H * DV), jnp.bfloat16),
        ],
    )(qkv, kv_a_ln_w.reshape(1, KVLR), W_kv_b, k_pe, cos2, sin2)


def _rope_i(x, c, s):
    rp = jnp.roll(x, 1, axis=1)
    rm = jnp.roll(x, -1, axis=1)
    lane = jax.lax.broadcasted_iota(jnp.int32, x.shape, 1)
    swap = jnp.where(lane % 2 == 0, -rm, rp)
    return x * c + swap * s


BMA = 512


def _attn_body(q_ref, kf_ref, v_ref, cos_ref, sin_ref, o_ref):
    qb = pl.program_id(1)
    q0 = qb * BMA
    cq = cos_ref[pl.ds(q0, BMA), :]
    sq = sin_ref[pl.ds(q0, BMA), :]
    qblk = q_ref[...]
    row = q0 + jax.lax.broadcasted_iota(jnp.int32, (BMA, BMA), 0)
    cloc = jax.lax.broadcasted_iota(jnp.int32, (BMA, BMA), 1)
    for hh in range(2):
        qh = qblk[:, hh * 192:(hh + 1) * 192]
        qf = jnp.concatenate(
            [qh[:, :DQN], _rope_i(qh[:, DQN:], cq, sq)], axis=1
        ).astype(jnp.bfloat16)

        def chunk(c, carry):
            m, l, acc = carry
            kf = kf_ref[pl.ds(c * BMA, BMA), hh * 192:(hh + 1) * 192]
            v = v_ref[pl.ds(c * BMA, BMA), hh * DV:(hh + 1) * DV]
            s = jax.lax.dot_general(
                qf, kf, (((1,), (1,)), ((), ())),
                preferred_element_type=jnp.float32) * SCALE
            s = jnp.where(c * BMA + cloc <= row, s, -1e30)
            mc = jnp.max(s, axis=1, keepdims=True)
            mn = jnp.maximum(m, mc)
            corr = jnp.exp(m - mn)
            p = jnp.exp(s - mn)
            ln = l * corr + jnp.sum(p, axis=1, keepdims=True)
            accn = acc * corr + jnp.dot(p.astype(jnp.bfloat16), v,
                                        preferred_element_type=jnp.float32)
            return mn, ln, accn

        m0 = jnp.full((BMA, 1), -jnp.inf, jnp.float32)
        l0 = jnp.zeros((BMA, 1), jnp.float32)
        a0 = jnp.zeros((BMA, DV), jnp.float32)
        _, l, acc = jax.lax.fori_loop(0, qb + 1, chunk, (m0, l0, a0))
        o_ref[:, hh * DV:(hh + 1) * DV] = (acc / l).astype(jnp.bfloat16)


def _attention(q, kf, v, cos2, sin2):
    return pl.pallas_call(
        _attn_body,
        grid=(H // 2, T // BMA),
        in_specs=[
            pl.BlockSpec((BMA, 384), lambda h, q: (q, h)),
            pl.BlockSpec((T, 384), lambda h, q: (0, h)),
            pl.BlockSpec((T, 2 * DV), lambda h, q: (0, h)),
            pl.BlockSpec((T, DR), lambda h, q: (0, 0)),
            pl.BlockSpec((T, DR), lambda h, q: (0, 0)),
        ],
        out_specs=pl.BlockSpec((BMA, 2 * DV), lambda h, q: (q, h)),
        out_shape=jax.ShapeDtypeStruct((T, H * DV), jnp.bfloat16),
    )(q, kf, v, cos2, sin2)


def _wo_body(o_ref, w_ref, hid_ref, h_ref):
    h_ref[...] = hid_ref[...] + jnp.dot(
        o_ref[...].astype(jnp.bfloat16), w_ref[...].astype(jnp.bfloat16),
        preferred_element_type=jnp.float32)


def _wo_res(o_cat, w_o, hidden):
    return pl.pallas_call(
        _wo_body,
        grid=(T // BM,),
        in_specs=[
            pl.BlockSpec((BM, H * DV), lambda i: (i, 0)),
            pl.BlockSpec((H * DV, D), lambda i: (0, 0)),
            pl.BlockSpec((BM, D), lambda i: (i, 0)),
        ],
        out_specs=pl.BlockSpec((BM, D), lambda i: (i, 0)),
        out_shape=jax.ShapeDtypeStruct((T, D), jnp.float32),
    )(o_cat, w_o, hidden)


def _router_body(h_ref, wln_ref, wr_ref, eb_ref, x2_ref, sel_ref, w_ref):
    h = h_ref[...]
    v = jnp.mean(h * h, axis=-1, keepdims=True)
    x2 = h * jax.lax.rsqrt(v + EPS) * wln_ref[...]
    x2_ref[...] = x2.astype(jnp.bfloat16)
    logits = jnp.dot(x2, wr_ref[...], preferred_element_type=jnp.float32)
    sc = jax.nn.sigmoid(logits)
    b = sc + eb_ref[...]
    idx = jax.lax.broadcasted_iota(jnp.int32, (BM, E), 1)
    sel = jnp.zeros((BM, E), jnp.float32)
    for _ in range(K):
        m = jnp.max(b, axis=1, keepdims=True)
        am = jnp.min(jnp.where(b == m, idx, E), axis=1, keepdims=True)
        pick = idx == am
        sel = jnp.where(pick, 1.0, sel)
        b = jnp.where(pick, -jnp.inf, b)
    sw = sel * sc
    denom = jnp.sum(sw, axis=1, keepdims=True) + 1e-20
    sel_ref[...] = sel
    w_ref[...] = sw / denom * RSF


def _router(h, post_ln_w, w_router, e_bias):
    return pl.pallas_call(
        _router_body,
        grid=(T // BM,),
        in_specs=[
            pl.BlockSpec((BM, D), lambda i: (i, 0)),
            pl.BlockSpec((1, D), lambda i: (0, 0)),
            pl.BlockSpec((D, E), lambda i: (0, 0)),
            pl.BlockSpec((1, E), lambda i: (0, 0)),
        ],
        out_specs=[
            pl.BlockSpec((BM, D), lambda i: (i, 0)),
            pl.BlockSpec((BM, E), lambda i: (i, 0)),
            pl.BlockSpec((BM, E), lambda i: (i, 0)),
        ],
        out_shape=[
            jax.ShapeDtypeStruct((T, D), jnp.bfloat16),
            jax.ShapeDtypeStruct((T, E), jnp.float32),
            jax.ShapeDtypeStruct((T, E), jnp.float32),
        ],
    )(h, post_ln_w.reshape(1, D), w_router, e_bias.reshape(1, E))


def _cap_body(sel_ref, w_ref, kw_ref, carry_ref):
    i = pl.program_id(0)

    @pl.when(i == 0)
    def _():
        carry_ref[...] = jnp.zeros_like(carry_ref)

    sel = sel_ref[...]
    r = jax.lax.broadcasted_iota(jnp.int32, (BM, BM), 0)
    c = jax.lax.broadcasted_iota(jnp.int32, (BM, BM), 1)
    tril = (r >= c).astype(jnp.bfloat16)
    cs = jnp.dot(tril, sel.astype(jnp.bfloat16),
                 preferred_element_type=jnp.float32)
    pie = cs - 1.0 + carry_ref[...]
    keep = jnp.where(pie < CAP, sel, 0.0)
    kw_ref[...] = w_ref[...] * keep
    carry_ref[...] += cs[BM - 1:BM, :]


def _capacity(sel, wfull):
    return pl.pallas_call(
        _cap_body,
        grid=(T // BM,),
        in_specs=[
            pl.BlockSpec((BM, E), lambda i: (i, 0)),
            pl.BlockSpec((BM, E), lambda i: (i, 0)),
        ],
        out_specs=pl.BlockSpec((BM, E), lambda i: (i, 0)),
        out_shape=jax.ShapeDtypeStruct((T, E), jnp.float32),
        scratch_shapes=[pltpu.VMEM((1, E), jnp.float32)],
    )(sel, wfull)


def _exp_body(x2_ref, wg_ref, wu_ref, kw_ref, hg_ref):
    e = pl.program_id(0)
    x2 = x2_ref[...]
    xg = jnp.dot(x2, wg_ref[0].astype(jnp.bfloat16),
                 preferred_element_type=jnp.float32)
    xu = jnp.dot(x2, wu_ref[0].astype(jnp.bfloat16),
                 preferred_element_type=jnp.float32)
    ohe = (jax.lax.broadcasted_iota(jnp.int32, (1, E), 1) == e
           ).astype(jnp.float32)
    kcol = jnp.sum(kw_ref[...] * ohe, axis=1, keepdims=True)
    hg = jax.nn.silu(xg) * xu * kcol
    hg_ref[...] = hg.astype(jnp.bfloat16)


def _experts_hg(x2b, w_g, w_u, keepw):
    return pl.pallas_call(
        _exp_body,
        grid=(E,),
        in_specs=[
            pl.BlockSpec((T, D), lambda e: (0, 0)),
            pl.BlockSpec((1, D, F), lambda e: (e, 0, 0)),
            pl.BlockSpec((1, D, F), lambda e: (e, 0, 0)),
            pl.BlockSpec((T, E), lambda e: (0, 0)),
        ],
        out_specs=pl.BlockSpec((T, F), lambda e: (0, e)),
        out_shape=jax.ShapeDtypeStruct((T, E * F), jnp.bfloat16),
    )(x2b, w_g, w_u, keepw)


def _final_body(hg_ref, wd_ref, h_ref, x2_ref, wsg_ref, wsu_ref, wsd_ref,
                o_ref):
    x2 = x2_ref[...]
    sg = jnp.dot(x2, wsg_ref[...].astype(jnp.bfloat16),
                 preferred_element_type=jnp.float32)
    su = jnp.dot(x2, wsu_ref[...].astype(jnp.bfloat16),
                 preferred_element_type=jnp.float32)
    hs = (jax.nn.silu(sg) * su).astype(jnp.bfloat16)
    shared = jnp.dot(hs, wsd_ref[...].astype(jnp.bfloat16),
                     preferred_element_type=jnp.float32)
    routed = jnp.dot(hg_ref[...], wd_ref[...],
                     preferred_element_type=jnp.float32)
    o_ref[...] = h_ref[...] + routed + shared


def _final(hg, wd_stack_bf16, h, x2b, ws_g, ws_u, ws_d):
    return pl.pallas_call(
        _final_body,
        grid=(T // BM,),
        in_specs=[
            pl.BlockSpec((BM, E * F), lambda i: (i, 0)),
            pl.BlockSpec((E * F, D), lambda i: (0, 0)),
            pl.BlockSpec((BM, D), lambda i: (i, 0)),
            pl.BlockSpec((BM, D), lambda i: (i, 0)),
            pl.BlockSpec((D, F), lambda i: (0, 0)),
            pl.BlockSpec((D, F), lambda i: (0, 0)),
            pl.BlockSpec((F, D), lambda i: (0, 0)),
        ],
        out_specs=pl.BlockSpec((BM, D), lambda i: (i, 0)),
        out_shape=jax.ShapeDtypeStruct((T, D), jnp.float32),
    )(hg, wd_stack_bf16, h, x2b, ws_g, ws_u, ws_d)


def kernel(hidden_states, positions, input_ln_w, post_ln_w, W_qkv_a,
           q_a_ln_w, W_q_b, kv_a_ln_w, W_kv_b, W_o, W_router, e_bias,
           W_g, W_u, W_d, Ws_g, Ws_u, Ws_d):
    inv = 1.0 / (10000.0 ** (jnp.arange(0, DR, 2, dtype=jnp.float32) / DR))
    ang = positions.astype(jnp.float32)[:, None] * inv[None, :]
    cos2 = jnp.repeat(jnp.cos(ang), 2, axis=1)
    sin2 = jnp.repeat(jnp.sin(ang), 2, axis=1)

    qkv = _mm_rms(hidden_states, input_ln_w, W_qkv_a)
    q = _mm_rms(qkv, q_a_ln_w, W_q_b, kwidth=QLR, colblk=0)
    kf, v = _kv_proj(qkv, kv_a_ln_w, W_kv_b, qkv[:, QLR + KVLR:],
                     cos2, sin2)

    o_cat = _attention(q, kf, v, cos2, sin2)
    h = _wo_res(o_cat, W_o, hidden_states)

    x2b, sel, wfull = _router(h, post_ln_w, W_router, e_bias)
    keepw = _capacity(sel, wfull)

    hg = _experts_hg(x2b, W_g, W_u, keepw)
    wd_stack = W_d.reshape(E * F, D).astype(jnp.bfloat16)
    out = _final(hg, wd_stack, h, x2b, Ws_g, Ws_u, Ws_d)
    return out


# speedup vs baseline: 1.6075x; 1.2267x over previous
import functools

import jax
import jax.numpy as jnp
from jax.experimental import pallas as pl
from jax.experimental.pallas import tpu as pltpu

T = 2048
D = 2048
H = 16
DQN = 128
DR = 64
DV = 128
QLR = 1536
KVLR = 512
E = 64
K = 8
F = 128
EPS = 1e-06
SCALE = (DQN + DR) ** -0.5
RSF = 2.5
CAP = int(T * K / E * 2)

BM = 256


def _rms_bf16(x, w):
    v = jnp.mean(x * x, axis=-1, keepdims=True)
    return (x * jax.lax.rsqrt(v + EPS) * w).astype(jnp.bfloat16)


def _mm_rms_body(x_ref, wln_ref, w_ref, o_ref):
    xn = _rms_bf16(x_ref[...], wln_ref[...])
    o_ref[...] = jnp.dot(xn, w_ref[...].astype(jnp.bfloat16),
                         preferred_element_type=jnp.float32)


def _mm_rms(x, wln, w, kwidth=None, colblk=0):
    t = x.shape[0]
    k = w.shape[0] if kwidth is None else kwidth
    n = w.shape[1]
    return pl.pallas_call(
        _mm_rms_body,
        grid=(t // BM,),
        in_specs=[
            pl.BlockSpec((BM, k), lambda i: (i, colblk)),
            pl.BlockSpec((1, k), lambda i: (0, 0)),
            pl.BlockSpec((k, n), lambda i: (0, 0)),
        ],
        out_specs=pl.BlockSpec((BM, n), lambda i: (i, 0)),
        out_shape=jax.ShapeDtypeStruct((t, n), jnp.float32),
    )(x, wln.reshape(1, k), w)


def _kv_body(x_ref, wln_ref, w_ref, kpe_ref, cos_ref, sin_ref,
             kf_ref, v_ref):
    xn = _rms_bf16(x_ref[...], wln_ref[...])
    kv = jnp.dot(xn, w_ref[...].astype(jnp.bfloat16),
                 preferred_element_type=jnp.float32)
    kpe = _rope_i(kpe_ref[...], cos_ref[...], sin_ref[...])
    kfs = []
    vs = []
    for h in range(H):
        kfs.append(kv[:, h * 256:h * 256 + DQN])
        kfs.append(kpe)
        vs.append(kv[:, h * 256 + DQN:(h + 1) * 256])
    kf_ref[...] = jnp.concatenate(kfs, axis=1).astype(jnp.bfloat16)
    v_ref[...] = jnp.concatenate(vs, axis=1).astype(jnp.bfloat16)


def _kv_proj(qkv, kv_a_ln_w, W_kv_b, k_pe, cos2, sin2):
    return pl.pallas_call(
        _kv_body,
        grid=(T // BM,),
        in_specs=[
            pl.BlockSpec((BM, KVLR), lambda i: (i, QLR // KVLR)),
            pl.BlockSpec((1, KVLR), lambda i: (0, 0)),
            pl.BlockSpec((KVLR, H * 256), lambda i: (0, 0)),
            pl.BlockSpec((BM, DR), lambda i: (i, 0)),
            pl.BlockSpec((BM, DR), lambda i: (i, 0)),
            pl.BlockSpec((BM, DR), lambda i: (i, 0)),
        ],
        out_specs=[
            pl.BlockSpec((BM, H * 192), lambda i: (i, 0)),
            pl.BlockSpec((BM, H * DV), lambda i: (i, 0)),
        ],
        out_shape=[
            jax.ShapeDtypeStruct((T, H * 192), jnp.bfloat16),
            jax.ShapeDtypeStruct((T, H * DV), jnp.bfloat16),
        ],
    )(qkv, kv_a_ln_w.reshape(1, KVLR), W_kv_b, k_pe, cos2, sin2)


def _rope_i(x, c, s):
    rp = jnp.roll(x, 1, axis=1)
    rm = jnp.roll(x, -1, axis=1)
    lane = jax.lax.broadcasted_iota(jnp.int32, x.shape, 1)
    swap = jnp.where(lane % 2 == 0, -rm, rp)
    return x * c + swap * s


BMA = 512


def _attn_body(q_ref, kf_ref, v_ref, cos_ref, sin_ref, o_ref):
    qb = pl.program_id(1)
    q0 = qb * BMA
    cq = cos_ref[pl.ds(q0, BMA), :]
    sq = sin_ref[pl.ds(q0, BMA), :]
    qblk = q_ref[...]
    row = q0 + jax.lax.broadcasted_iota(jnp.int32, (BMA, BMA), 0)
    cloc = jax.lax.broadcasted_iota(jnp.int32, (BMA, BMA), 1)
    for hh in range(2):
        qh = qblk[:, hh * 192:(hh + 1) * 192]
        qf = jnp.concatenate(
            [qh[:, :DQN], _rope_i(qh[:, DQN:], cq, sq)], axis=1
        ).astype(jnp.bfloat16)

        def chunk(c, carry):
            m, l, acc = carry
            kf = kf_ref[pl.ds(c * BMA, BMA), hh * 192:(hh + 1) * 192]
            v = v_ref[pl.ds(c * BMA, BMA), hh * DV:(hh + 1) * DV]
            s = jax.lax.dot_general(
                qf, kf, (((1,), (1,)), ((), ())),
                preferred_element_type=jnp.float32) * SCALE
            s = jnp.where(c * BMA + cloc <= row, s, -1e30)
            mc = jnp.max(s, axis=1, keepdims=True)
            mn = jnp.maximum(m, mc)
            corr = jnp.exp(m - mn)
            p = jnp.exp(s - mn)
            ln = l * corr + jnp.sum(p, axis=1, keepdims=True)
            accn = acc * corr + jnp.dot(p.astype(jnp.bfloat16), v,
                                        preferred_element_type=jnp.float32)
            return mn, ln, accn

        m0 = jnp.full((BMA, 1), -jnp.inf, jnp.float32)
        l0 = jnp.zeros((BMA, 1), jnp.float32)
        a0 = jnp.zeros((BMA, DV), jnp.float32)
        _, l, acc = jax.lax.fori_loop(0, qb + 1, chunk, (m0, l0, a0))
        o_ref[:, hh * DV:(hh + 1) * DV] = (acc / l).astype(jnp.bfloat16)


def _attention(q, kf, v, cos2, sin2):
    return pl.pallas_call(
        _attn_body,
        grid=(H // 2, T // BMA),
        in_specs=[
            pl.BlockSpec((BMA, 384), lambda h, q: (q, h)),
            pl.BlockSpec((T, 384), lambda h, q: (0, h)),
            pl.BlockSpec((T, 2 * DV), lambda h, q: (0, h)),
            pl.BlockSpec((T, DR), lambda h, q: (0, 0)),
            pl.BlockSpec((T, DR), lambda h, q: (0, 0)),
        ],
        out_specs=pl.BlockSpec((BMA, 2 * DV), lambda h, q: (q, h)),
        out_shape=jax.ShapeDtypeStruct((T, H * DV), jnp.bfloat16),
    )(q, kf, v, cos2, sin2)


def _route_body(o_ref, wo_ref, hid_ref, pln_ref, wr_ref, eb_ref,
                h_ref, x2_ref, kw_ref, carry_ref):
    i = pl.program_id(0)

    @pl.when(i == 0)
    def _():
        carry_ref[...] = jnp.zeros_like(carry_ref)

    h = hid_ref[...] + jnp.dot(
        o_ref[...], wo_ref[...].astype(jnp.bfloat16),
        preferred_element_type=jnp.float32)
    h_ref[...] = h
    vr = jnp.mean(h * h, axis=-1, keepdims=True)
    x2 = h * jax.lax.rsqrt(vr + EPS) * pln_ref[...]
    x2_ref[...] = x2.astype(jnp.bfloat16)
    logits = jnp.dot(x2, wr_ref[...], preferred_element_type=jnp.float32)
    sc = jax.nn.sigmoid(logits)
    b = sc + eb_ref[...]
    idx = jax.lax.broadcasted_iota(jnp.int32, (BM, E), 1)
    sel = jnp.zeros((BM, E), jnp.float32)
    for _ in range(K):
        m = jnp.max(b, axis=1, keepdims=True)
        am = jnp.min(jnp.where(b == m, idx, E), axis=1, keepdims=True)
        pick = idx == am
        sel = jnp.where(pick, 1.0, sel)
        b = jnp.where(pick, -jnp.inf, b)
    sw = sel * sc
    denom = jnp.sum(sw, axis=1, keepdims=True) + 1e-20
    wfull = sw / denom * RSF
    r = jax.lax.broadcasted_iota(jnp.int32, (BM, BM), 0)
    c = jax.lax.broadcasted_iota(jnp.int32, (BM, BM), 1)
    tril = (r >= c).astype(jnp.bfloat16)
    cs = jnp.dot(tril, sel.astype(jnp.bfloat16),
                 preferred_element_type=jnp.float32)
    pie = cs - 1.0 + carry_ref[...]
    keep = jnp.where(pie < CAP, sel, 0.0)
    kw_ref[...] = wfull * keep
    carry_ref[...] += cs[BM - 1:BM, :]


def _route(o_cat, w_o, hidden, post_ln_w, w_router, e_bias):
    return pl.pallas_call(
        _route_body,
        grid=(T // BM,),
        in_specs=[
            pl.BlockSpec((BM, H * DV), lambda i: (i, 0)),
            pl.BlockSpec((H * DV, D), lambda i: (0, 0)),
            pl.BlockSpec((BM, D), lambda i: (i, 0)),
            pl.BlockSpec((1, D), lambda i: (0, 0)),
            pl.BlockSpec((D, E), lambda i: (0, 0)),
            pl.BlockSpec((1, E), lambda i: (0, 0)),
        ],
        out_specs=[
            pl.BlockSpec((BM, D), lambda i: (i, 0)),
            pl.BlockSpec((BM, D), lambda i: (i, 0)),
            pl.BlockSpec((BM, E), lambda i: (i, 0)),
        ],
        out_shape=[
            jax.ShapeDtypeStruct((T, D), jnp.float32),
            jax.ShapeDtypeStruct((T, D), jnp.bfloat16),
            jax.ShapeDtypeStruct((T, E), jnp.float32),
        ],
        scratch_shapes=[pltpu.VMEM((1, E), jnp.float32)],
    )(o_cat, w_o, hidden, post_ln_w.reshape(1, D), w_router,
      e_bias.reshape(1, E))


EPP = 4


def _exp_body(x2_ref, wg_ref, wu_ref, kw_ref, hg_ref):
    pid = pl.program_id(0)
    x2 = x2_ref[...]
    wg = jnp.concatenate([wg_ref[j] for j in range(EPP)],
                         axis=1).astype(jnp.bfloat16)
    wu = jnp.concatenate([wu_ref[j] for j in range(EPP)],
                         axis=1).astype(jnp.bfloat16)
    xg = jnp.dot(x2, wg, preferred_element_type=jnp.float32)
    xu = jnp.dot(x2, wu, preferred_element_type=jnp.float32)
    iot = jax.lax.broadcasted_iota(jnp.int32, (1, E), 1)
    kw = kw_ref[...]
    scale = jnp.concatenate(
        [jnp.broadcast_to(
            jnp.sum(kw * (iot == pid * EPP + j).astype(jnp.float32),
                    axis=1, keepdims=True), (T, F))
         for j in range(EPP)], axis=1)
    hg_ref[...] = (jax.nn.silu(xg) * xu * scale).astype(jnp.bfloat16)


def _experts_hg(x2b, w_g, w_u, keepw):
    return pl.pallas_call(
        _exp_body,
        grid=(E // EPP,),
        in_specs=[
            pl.BlockSpec((T, D), lambda e: (0, 0)),
            pl.BlockSpec((EPP, D, F), lambda e: (e, 0, 0)),
            pl.BlockSpec((EPP, D, F), lambda e: (e, 0, 0)),
            pl.BlockSpec((T, E), lambda e: (0, 0)),
        ],
        out_specs=pl.BlockSpec((T, EPP * F), lambda e: (0, e)),
        out_shape=jax.ShapeDtypeStruct((T, E * F), jnp.bfloat16),
    )(x2b, w_g, w_u, keepw)


def _final_body(hg_ref, wd_ref, h_ref, x2_ref, wsg_ref, wsu_ref, wsd_ref,
                o_ref):
    x2 = x2_ref[...]
    sg = jnp.dot(x2, wsg_ref[...].astype(jnp.bfloat16),
                 preferred_element_type=jnp.float32)
    su = jnp.dot(x2, wsu_ref[...].astype(jnp.bfloat16),
                 preferred_element_type=jnp.float32)
    hs = (jax.nn.silu(sg) * su).astype(jnp.bfloat16)
    shared = jnp.dot(hs, wsd_ref[...].astype(jnp.bfloat16),
                     preferred_element_type=jnp.float32)
    routed = jnp.dot(hg_ref[...], wd_ref[...],
                     preferred_element_type=jnp.float32)
    o_ref[...] = h_ref[...] + routed + shared


def _final(hg, wd_stack_bf16, h, x2b, ws_g, ws_u, ws_d):
    return pl.pallas_call(
        _final_body,
        grid=(T // BM,),
        in_specs=[
            pl.BlockSpec((BM, E * F), lambda i: (i, 0)),
            pl.BlockSpec((E * F, D), lambda i: (0, 0)),
            pl.BlockSpec((BM, D), lambda i: (i, 0)),
            pl.BlockSpec((BM, D), lambda i: (i, 0)),
            pl.BlockSpec((D, F), lambda i: (0, 0)),
            pl.BlockSpec((D, F), lambda i: (0, 0)),
            pl.BlockSpec((F, D), lambda i: (0, 0)),
        ],
        out_specs=pl.BlockSpec((BM, D), lambda i: (i, 0)),
        out_shape=jax.ShapeDtypeStruct((T, D), jnp.float32),
    )(hg, wd_stack_bf16, h, x2b, ws_g, ws_u, ws_d)


def kernel(hidden_states, positions, input_ln_w, post_ln_w, W_qkv_a,
           q_a_ln_w, W_q_b, kv_a_ln_w, W_kv_b, W_o, W_router, e_bias,
           W_g, W_u, W_d, Ws_g, Ws_u, Ws_d):
    inv = 1.0 / (10000.0 ** (jnp.arange(0, DR, 2, dtype=jnp.float32) / DR))
    ang = positions.astype(jnp.float32)[:, None] * inv[None, :]
    cos2 = jnp.repeat(jnp.cos(ang), 2, axis=1)
    sin2 = jnp.repeat(jnp.sin(ang), 2, axis=1)

    qkv = _mm_rms(hidden_states, input_ln_w, W_qkv_a)
    q = _mm_rms(qkv, q_a_ln_w, W_q_b, kwidth=QLR, colblk=0)
    kf, v = _kv_proj(qkv, kv_a_ln_w, W_kv_b, qkv[:, QLR + KVLR:],
                     cos2, sin2)

    o_cat = _attention(q, kf, v, cos2, sin2)
    h, x2b, keepw = _route(o_cat, W_o, hidden_states, post_ln_w,
                           W_router, e_bias)

    hg = _experts_hg(x2b, W_g, W_u, keepw)
    wd_stack = W_d.reshape(E * F, D).astype(jnp.bfloat16)
    out = _final(hg, wd_stack, h, x2b, Ws_g, Ws_u, Ws_d)
    return out


# speedup vs baseline: 1.7461x; 1.0863x over previous
import functools

import jax
import jax.numpy as jnp
from jax.experimental import pallas as pl
from jax.experimental.pallas import tpu as pltpu

T = 2048
D = 2048
H = 16
DQN = 128
DR = 64
DV = 128
QLR = 1536
KVLR = 512
E = 64
K = 8
F = 128
EPS = 1e-06
SCALE = (DQN + DR) ** -0.5
RSF = 2.5
CAP = int(T * K / E * 2)

BM = 256


def _rms_bf16(x, w):
    v = jnp.mean(x * x, axis=-1, keepdims=True)
    return (x * jax.lax.rsqrt(v + EPS) * w).astype(jnp.bfloat16)


def _pre_body(hid_ref, iln_ref, wa_ref, qln_ref, wqb_ref, kvln_ref,
              wkvb_ref, cos_ref, sin_ref, q_ref, kf_ref, v_ref):
    xn = _rms_bf16(hid_ref[...], iln_ref[...])
    qkv = jnp.dot(xn, wa_ref[...].astype(jnp.bfloat16),
                  preferred_element_type=jnp.float32)
    qn_l = _rms_bf16(qkv[:, :QLR], qln_ref[...])
    q_ref[...] = jnp.dot(qn_l, wqb_ref[...].astype(jnp.bfloat16),
                         preferred_element_type=jnp.float32
                         ).astype(jnp.bfloat16)
    kvn = _rms_bf16(qkv[:, QLR:QLR + KVLR], kvln_ref[...])
    kv = jnp.dot(kvn, wkvb_ref[...].astype(jnp.bfloat16),
                 preferred_element_type=jnp.float32)
    kpe = _rope_i(qkv[:, QLR + KVLR:], cos_ref[...], sin_ref[...])
    kfs = []
    vs = []
    for h in range(H):
        kfs.append(kv[:, h * 256:h * 256 + DQN])
        kfs.append(kpe)
        vs.append(kv[:, h * 256 + DQN:(h + 1) * 256])
    kf_ref[...] = jnp.concatenate(kfs, axis=1).astype(jnp.bfloat16)
    v_ref[...] = jnp.concatenate(vs, axis=1).astype(jnp.bfloat16)


def _pre(hidden, input_ln_w, W_qkv_a, q_a_ln_w, W_q_b, kv_a_ln_w,
         W_kv_b, cos2, sin2):
    na = QLR + KVLR + DR
    return pl.pallas_call(
        _pre_body,
        grid=(T // BM,),
        in_specs=[
            pl.BlockSpec((BM, D), lambda i: (i, 0)),
            pl.BlockSpec((1, D), lambda i: (0, 0)),
            pl.BlockSpec((D, na), lambda i: (0, 0)),
            pl.BlockSpec((1, QLR), lambda i: (0, 0)),
            pl.BlockSpec((QLR, H * 192), lambda i: (0, 0)),
            pl.BlockSpec((1, KVLR), lambda i: (0, 0)),
            pl.BlockSpec((KVLR, H * 256), lambda i: (0, 0)),
            pl.BlockSpec((BM, DR), lambda i: (i, 0)),
            pl.BlockSpec((BM, DR), lambda i: (i, 0)),
        ],
        out_specs=[
            pl.BlockSpec((BM, H * 192), lambda i: (i, 0)),
            pl.BlockSpec((BM, H * 192), lambda i: (i, 0)),
            pl.BlockSpec((BM, H * DV), lambda i: (i, 0)),
        ],
        out_shape=[
            jax.ShapeDtypeStruct((T, H * 192), jnp.bfloat16),
            jax.ShapeDtypeStruct((T, H * 192), jnp.bfloat16),
            jax.ShapeDtypeStruct((T, H * DV), jnp.bfloat16),
        ],
    )(hidden, input_ln_w.reshape(1, D), W_qkv_a,
      q_a_ln_w.reshape(1, QLR), W_q_b, kv_a_ln_w.reshape(1, KVLR),
      W_kv_b, cos2, sin2)


def _rope_i(x, c, s):
    rp = jnp.roll(x, 1, axis=1)
    rm = jnp.roll(x, -1, axis=1)
    lane = jax.lax.broadcasted_iota(jnp.int32, x.shape, 1)
    swap = jnp.where(lane % 2 == 0, -rm, rp)
    return x * c + swap * s


BMA = 512


def _attn_body(q_ref, kf_ref, v_ref, cos_ref, sin_ref, o_ref):
    qb = pl.program_id(1)
    q0 = qb * BMA
    cq = cos_ref[pl.ds(q0, BMA), :]
    sq = sin_ref[pl.ds(q0, BMA), :]
    qblk = q_ref[...]
    row = q0 + jax.lax.broadcasted_iota(jnp.int32, (BMA, BMA), 0)
    cloc = jax.lax.broadcasted_iota(jnp.int32, (BMA, BMA), 1)
    for hh in range(2):
        qh = qblk[:, hh * 192:(hh + 1) * 192]
        qf = jnp.concatenate(
            [qh[:, :DQN],
             _rope_i(qh[:, DQN:], cq, sq).astype(jnp.bfloat16)], axis=1)

        def chunk(c, carry):
            m, l, acc = carry
            kf = kf_ref[pl.ds(c * BMA, BMA), hh * 192:(hh + 1) * 192]
            v = v_ref[pl.ds(c * BMA, BMA), hh * DV:(hh + 1) * DV]
            s = jax.lax.dot_general(
                qf, kf, (((1,), (1,)), ((), ())),
                preferred_element_type=jnp.float32) * SCALE
            s = jnp.where(c * BMA + cloc <= row, s, -1e30)
            mc = jnp.max(s, axis=1, keepdims=True)
            mn = jnp.maximum(m, mc)
            corr = jnp.exp(m - mn)
            p = jnp.exp(s - mn)
            ln = l * corr + jnp.sum(p, axis=1, keepdims=True)
            accn = acc * corr + jnp.dot(p.astype(jnp.bfloat16), v,
                                        preferred_element_type=jnp.float32)
            return mn, ln, accn

        m0 = jnp.full((BMA, 1), -jnp.inf, jnp.float32)
        l0 = jnp.zeros((BMA, 1), jnp.float32)
        a0 = jnp.zeros((BMA, DV), jnp.float32)
        _, l, acc = jax.lax.fori_loop(0, qb + 1, chunk, (m0, l0, a0))
        o_ref[:, hh * DV:(hh + 1) * DV] = (acc / l).astype(jnp.bfloat16)


def _attention(q, kf, v, cos2, sin2):
    return pl.pallas_call(
        _attn_body,
        grid=(H // 2, T // BMA),
        in_specs=[
            pl.BlockSpec((BMA, 384), lambda h, q: (q, h)),
            pl.BlockSpec((T, 384), lambda h, q: (0, h)),
            pl.BlockSpec((T, 2 * DV), lambda h, q: (0, h)),
            pl.BlockSpec((T, DR), lambda h, q: (0, 0)),
            pl.BlockSpec((T, DR), lambda h, q: (0, 0)),
        ],
        out_specs=pl.BlockSpec((BMA, 2 * DV), lambda h, q: (q, h)),
        out_shape=jax.ShapeDtypeStruct((T, H * DV), jnp.bfloat16),
    )(q, kf, v, cos2, sin2)


def _route_body(o_ref, wo_ref, hid_ref, pln_ref, wr_ref, eb_ref,
                h_ref, x2_ref, kw_ref, carry_ref):
    i = pl.program_id(0)

    @pl.when(i == 0)
    def _():
        carry_ref[...] = jnp.zeros_like(carry_ref)

    h = hid_ref[...] + jnp.dot(
        o_ref[...], wo_ref[...].astype(jnp.bfloat16),
        preferred_element_type=jnp.float32)
    h_ref[...] = h
    vr = jnp.mean(h * h, axis=-1, keepdims=True)
    x2 = h * jax.lax.rsqrt(vr + EPS) * pln_ref[...]
    x2_ref[...] = x2.astype(jnp.bfloat16)
    logits = jnp.dot(x2, wr_ref[...], preferred_element_type=jnp.float32)
    sc = jax.nn.sigmoid(logits)
    b = sc + eb_ref[...]
    idx = jax.lax.broadcasted_iota(jnp.int32, (BM, E), 1)
    sel = jnp.zeros((BM, E), jnp.float32)
    for _ in range(K):
        m = jnp.max(b, axis=1, keepdims=True)
        am = jnp.min(jnp.where(b == m, idx, E), axis=1, keepdims=True)
        pick = idx == am
        sel = jnp.where(pick, 1.0, sel)
        b = jnp.where(pick, -jnp.inf, b)
    sw = sel * sc
    denom = jnp.sum(sw, axis=1, keepdims=True) + 1e-20
    wfull = sw / denom * RSF
    r = jax.lax.broadcasted_iota(jnp.int32, (BM, BM), 0)
    c = jax.lax.broadcasted_iota(jnp.int32, (BM, BM), 1)
    tril = (r >= c).astype(jnp.bfloat16)
    cs = jnp.dot(tril, sel.astype(jnp.bfloat16),
                 preferred_element_type=jnp.float32)
    pie = cs - 1.0 + carry_ref[...]
    keep = jnp.where(pie < CAP, sel, 0.0)
    kw_ref[...] = wfull * keep
    carry_ref[...] += cs[BM - 1:BM, :]


def _route(o_cat, w_o, hidden, post_ln_w, w_router, e_bias):
    return pl.pallas_call(
        _route_body,
        grid=(T // BM,),
        in_specs=[
            pl.BlockSpec((BM, H * DV), lambda i: (i, 0)),
            pl.BlockSpec((H * DV, D), lambda i: (0, 0)),
            pl.BlockSpec((BM, D), lambda i: (i, 0)),
            pl.BlockSpec((1, D), lambda i: (0, 0)),
            pl.BlockSpec((D, E), lambda i: (0, 0)),
            pl.BlockSpec((1, E), lambda i: (0, 0)),
        ],
        out_specs=[
            pl.BlockSpec((BM, D), lambda i: (i, 0)),
            pl.BlockSpec((BM, D), lambda i: (i, 0)),
            pl.BlockSpec((BM, E), lambda i: (i, 0)),
        ],
        out_shape=[
            jax.ShapeDtypeStruct((T, D), jnp.float32),
            jax.ShapeDtypeStruct((T, D), jnp.bfloat16),
            jax.ShapeDtypeStruct((T, E), jnp.float32),
        ],
        scratch_shapes=[pltpu.VMEM((1, E), jnp.float32)],
    )(o_cat, w_o, hidden, post_ln_w.reshape(1, D), w_router,
      e_bias.reshape(1, E))


EPP = 4


def _exp_body(x2_ref, wg_ref, wu_ref, kw_ref, wd_ref, hg_ref, wds_ref):
    pid = pl.program_id(0)
    wds_ref[...] = wd_ref[...].reshape(EPP * F, D).astype(jnp.bfloat16)
    x2 = x2_ref[...]
    wg = jnp.concatenate([wg_ref[j] for j in range(EPP)],
                         axis=1).astype(jnp.bfloat16)
    wu = jnp.concatenate([wu_ref[j] for j in range(EPP)],
                         axis=1).astype(jnp.bfloat16)
    xg = jnp.dot(x2, wg, preferred_element_type=jnp.float32)
    xu = jnp.dot(x2, wu, preferred_element_type=jnp.float32)
    iot = jax.lax.broadcasted_iota(jnp.int32, (1, E), 1)
    kw = kw_ref[...]
    scale = jnp.concatenate(
        [jnp.broadcast_to(
            jnp.sum(kw * (iot == pid * EPP + j).astype(jnp.float32),
                    axis=1, keepdims=True), (T, F))
         for j in range(EPP)], axis=1)
    hg_ref[...] = (jax.nn.silu(xg) * xu * scale).astype(jnp.bfloat16)


def _experts_hg(x2b, w_g, w_u, keepw, w_d):
    return pl.pallas_call(
        _exp_body,
        grid=(E // EPP,),
        in_specs=[
            pl.BlockSpec((T, D), lambda e: (0, 0)),
            pl.BlockSpec((EPP, D, F), lambda e: (e, 0, 0)),
            pl.BlockSpec((EPP, D, F), lambda e: (e, 0, 0)),
            pl.BlockSpec((T, E), lambda e: (0, 0)),
            pl.BlockSpec((EPP, F, D), lambda e: (e, 0, 0)),
        ],
        out_specs=[
            pl.BlockSpec((T, EPP * F), lambda e: (0, e)),
            pl.BlockSpec((EPP * F, D), lambda e: (e, 0)),
        ],
        out_shape=[
            jax.ShapeDtypeStruct((T, E * F), jnp.bfloat16),
            jax.ShapeDtypeStruct((E * F, D), jnp.bfloat16),
        ],
    )(x2b, w_g, w_u, keepw, w_d)


def _final_body(hg_ref, wd_ref, h_ref, x2_ref, wsg_ref, wsu_ref, wsd_ref,
                o_ref):
    x2 = x2_ref[...]
    sg = jnp.dot(x2, wsg_ref[...].astype(jnp.bfloat16),
                 preferred_element_type=jnp.float32)
    su = jnp.dot(x2, wsu_ref[...].astype(jnp.bfloat16),
                 preferred_element_type=jnp.float32)
    hs = (jax.nn.silu(sg) * su).astype(jnp.bfloat16)
    shared = jnp.dot(hs, wsd_ref[...].astype(jnp.bfloat16),
                     preferred_element_type=jnp.float32)
    routed = jnp.dot(hg_ref[...], wd_ref[...],
                     preferred_element_type=jnp.float32)
    o_ref[...] = h_ref[...] + routed + shared


def _final(hg, wd_stack_bf16, h, x2b, ws_g, ws_u, ws_d):
    return pl.pallas_call(
        _final_body,
        grid=(T // BM,),
        in_specs=[
            pl.BlockSpec((BM, E * F), lambda i: (i, 0)),
            pl.BlockSpec((E * F, D), lambda i: (0, 0)),
            pl.BlockSpec((BM, D), lambda i: (i, 0)),
            pl.BlockSpec((BM, D), lambda i: (i, 0)),
            pl.BlockSpec((D, F), lambda i: (0, 0)),
            pl.BlockSpec((D, F), lambda i: (0, 0)),
            pl.BlockSpec((F, D), lambda i: (0, 0)),
        ],
        out_specs=pl.BlockSpec((BM, D), lambda i: (i, 0)),
        out_shape=jax.ShapeDtypeStruct((T, D), jnp.float32),
    )(hg, wd_stack_bf16, h, x2b, ws_g, ws_u, ws_d)


def kernel(hidden_states, positions, input_ln_w, post_ln_w, W_qkv_a,
           q_a_ln_w, W_q_b, kv_a_ln_w, W_kv_b, W_o, W_router, e_bias,
           W_g, W_u, W_d, Ws_g, Ws_u, Ws_d):
    inv = 1.0 / (10000.0 ** (jnp.arange(0, DR, 2, dtype=jnp.float32) / DR))
    ang = positions.astype(jnp.float32)[:, None] * inv[None, :]
    cos2 = jnp.repeat(jnp.cos(ang), 2, axis=1)
    sin2 = jnp.repeat(jnp.sin(ang), 2, axis=1)

    q, kf, v = _pre(hidden_states, input_ln_w, W_qkv_a, q_a_ln_w, W_q_b,
                    kv_a_ln_w, W_kv_b, cos2, sin2)

    o_cat = _attention(q, kf, v, cos2, sin2)
    h, x2b, keepw = _route(o_cat, W_o, hidden_states, post_ln_w,
                           W_router, e_bias)

    hg, wd_stack = _experts_hg(x2b, W_g, W_u, keepw, W_d)
    out = _final(hg, wd_stack, h, x2b, Ws_g, Ws_u, Ws_d)
    return out


# speedup vs baseline: 1.9945x; 1.1422x over previous
import functools

import jax
import jax.numpy as jnp
from jax.experimental import pallas as pl
from jax.experimental.pallas import tpu as pltpu

T = 2048
D = 2048
H = 16
DQN = 128
DR = 64
DV = 128
QLR = 1536
KVLR = 512
E = 64
K = 8
F = 128
EPS = 1e-06
SCALE = (DQN + DR) ** -0.5
RSF = 2.5
CAP = int(T * K / E * 2)

BM = 256


def _rms_bf16(x, w):
    v = jnp.mean(x * x, axis=-1, keepdims=True)
    return (x * jax.lax.rsqrt(v + EPS) * w).astype(jnp.bfloat16)


def _pre_body(hid_ref, iln_ref, wa_ref, qln_ref, wqb_ref, kvln_ref,
              wkvb_ref, cos_ref, sin_ref, q_ref, kf_ref, v_ref):
    xn = _rms_bf16(hid_ref[...], iln_ref[...])
    qkv = jnp.dot(xn, wa_ref[...].astype(jnp.bfloat16),
                  preferred_element_type=jnp.float32)
    qn_l = _rms_bf16(qkv[:, :QLR], qln_ref[...])
    q_ref[...] = jnp.dot(qn_l, wqb_ref[...].astype(jnp.bfloat16),
                         preferred_element_type=jnp.float32
                         ).astype(jnp.bfloat16)
    kvn = _rms_bf16(qkv[:, QLR:QLR + KVLR], kvln_ref[...])
    kv = jnp.dot(kvn, wkvb_ref[...].astype(jnp.bfloat16),
                 preferred_element_type=jnp.float32)
    kpe = _rope_i(qkv[:, QLR + KVLR:], cos_ref[...], sin_ref[...])
    kfs = []
    vs = []
    for h in range(H):
        kfs.append(kv[:, h * 256:h * 256 + DQN])
        kfs.append(kpe)
        vs.append(kv[:, h * 256 + DQN:(h + 1) * 256])
    kf_ref[...] = jnp.concatenate(kfs, axis=1).astype(jnp.bfloat16)
    v_ref[...] = jnp.concatenate(vs, axis=1).astype(jnp.bfloat16)


def _pre(hidden, input_ln_w, W_qkv_a, q_a_ln_w, W_q_b, kv_a_ln_w,
         W_kv_b, cos2, sin2):
    na = QLR + KVLR + DR
    return pl.pallas_call(
        _pre_body,
        grid=(T // BM,),
        in_specs=[
            pl.BlockSpec((BM, D), lambda i: (i, 0)),
            pl.BlockSpec((1, D), lambda i: (0, 0)),
            pl.BlockSpec((D, na), lambda i: (0, 0)),
            pl.BlockSpec((1, QLR), lambda i: (0, 0)),
            pl.BlockSpec((QLR, H * 192), lambda i: (0, 0)),
            pl.BlockSpec((1, KVLR), lambda i: (0, 0)),
            pl.BlockSpec((KVLR, H * 256), lambda i: (0, 0)),
            pl.BlockSpec((BM, DR), lambda i: (i, 0)),
            pl.BlockSpec((BM, DR), lambda i: (i, 0)),
        ],
        out_specs=[
            pl.BlockSpec((BM, H * 192), lambda i: (i, 0)),
            pl.BlockSpec((BM, H * 192), lambda i: (i, 0)),
            pl.BlockSpec((BM, H * DV), lambda i: (i, 0)),
        ],
        out_shape=[
            jax.ShapeDtypeStruct((T, H * 192), jnp.bfloat16),
            jax.ShapeDtypeStruct((T, H * 192), jnp.bfloat16),
            jax.ShapeDtypeStruct((T, H * DV), jnp.bfloat16),
        ],
    )(hidden, input_ln_w.reshape(1, D), W_qkv_a,
      q_a_ln_w.reshape(1, QLR), W_q_b, kv_a_ln_w.reshape(1, KVLR),
      W_kv_b, cos2, sin2)


def _rope_i(x, c, s):
    rp = jnp.roll(x, 1, axis=1)
    rm = jnp.roll(x, -1, axis=1)
    lane = jax.lax.broadcasted_iota(jnp.int32, x.shape, 1)
    swap = jnp.where(lane % 2 == 0, -rm, rp)
    return x * c + swap * s


BMA = 512


def _attn_body(q_ref, kf_ref, v_ref, cos_ref, sin_ref, o_ref):
    qb = pl.program_id(1)
    q0 = qb * BMA
    cq = cos_ref[pl.ds(q0, BMA), :]
    sq = sin_ref[pl.ds(q0, BMA), :]
    qblk = q_ref[...]
    row = q0 + jax.lax.broadcasted_iota(jnp.int32, (BMA, BMA), 0)
    cloc = jax.lax.broadcasted_iota(jnp.int32, (BMA, BMA), 1)
    for hh in range(2):
        qh = qblk[:, hh * 192:(hh + 1) * 192]
        qf = jnp.concatenate(
            [qh[:, :DQN],
             _rope_i(qh[:, DQN:], cq, sq).astype(jnp.bfloat16)], axis=1)

        def chunk(c, carry):
            m, l, acc = carry
            kf = kf_ref[pl.ds(c * BMA, BMA), hh * 192:(hh + 1) * 192]
            v = v_ref[pl.ds(c * BMA, BMA), hh * DV:(hh + 1) * DV]
            s = jax.lax.dot_general(
                qf, kf, (((1,), (1,)), ((), ())),
                preferred_element_type=jnp.float32) * SCALE
            s = jnp.where(c * BMA + cloc <= row, s, -1e30)
            mc = jnp.max(s, axis=1, keepdims=True)
            mn = jnp.maximum(m, mc)
            corr = jnp.exp(m - mn)
            p = jnp.exp(s - mn)
            ln = l * corr + jnp.sum(p, axis=1, keepdims=True)
            accn = acc * corr + jnp.dot(p.astype(jnp.bfloat16), v,
                                        preferred_element_type=jnp.float32)
            return mn, ln, accn

        m0 = jnp.full((BMA, 1), -jnp.inf, jnp.float32)
        l0 = jnp.zeros((BMA, 1), jnp.float32)
        a0 = jnp.zeros((BMA, DV), jnp.float32)
        _, l, acc = jax.lax.fori_loop(0, qb + 1, chunk, (m0, l0, a0))
        o_ref[:, hh * DV:(hh + 1) * DV] = (acc / l).astype(jnp.bfloat16)


def _attention(q, kf, v, cos2, sin2):
    return pl.pallas_call(
        _attn_body,
        grid=(H // 2, T // BMA),
        in_specs=[
            pl.BlockSpec((BMA, 384), lambda h, q: (q, h)),
            pl.BlockSpec((T, 384), lambda h, q: (0, h)),
            pl.BlockSpec((T, 2 * DV), lambda h, q: (0, h)),
            pl.BlockSpec((T, DR), lambda h, q: (0, 0)),
            pl.BlockSpec((T, DR), lambda h, q: (0, 0)),
        ],
        out_specs=pl.BlockSpec((BMA, 2 * DV), lambda h, q: (q, h)),
        out_shape=jax.ShapeDtypeStruct((T, H * DV), jnp.bfloat16),
    )(q, kf, v, cos2, sin2)


def _route_body(o_ref, wo_ref, hid_ref, pln_ref, wr_ref, eb_ref,
                h_ref, x2_ref, kw_ref, carry_ref):
    i = pl.program_id(0)

    @pl.when(i == 0)
    def _():
        carry_ref[...] = jnp.zeros_like(carry_ref)

    h = hid_ref[...] + jnp.dot(
        o_ref[...], wo_ref[...].astype(jnp.bfloat16),
        preferred_element_type=jnp.float32)
    h_ref[...] = h
    vr = jnp.mean(h * h, axis=-1, keepdims=True)
    x2 = h * jax.lax.rsqrt(vr + EPS) * pln_ref[...]
    x2_ref[...] = x2.astype(jnp.bfloat16)
    logits = jnp.dot(x2, wr_ref[...], preferred_element_type=jnp.float32)
    sc = jax.nn.sigmoid(logits)
    b = sc + eb_ref[...]
    idx = jax.lax.broadcasted_iota(jnp.int32, (BM, E), 1)
    sel = jnp.zeros((BM, E), jnp.float32)
    for _ in range(K):
        m = jnp.max(b, axis=1, keepdims=True)
        am = jnp.min(jnp.where(b == m, idx, E), axis=1, keepdims=True)
        pick = idx == am
        sel = jnp.where(pick, 1.0, sel)
        b = jnp.where(pick, -jnp.inf, b)
    sw = sel * sc
    denom = jnp.sum(sw, axis=1, keepdims=True) + 1e-20
    wfull = sw / denom * RSF
    r = jax.lax.broadcasted_iota(jnp.int32, (BM, BM), 0)
    c = jax.lax.broadcasted_iota(jnp.int32, (BM, BM), 1)
    tril = (r >= c).astype(jnp.bfloat16)
    cs = jnp.dot(tril, sel.astype(jnp.bfloat16),
                 preferred_element_type=jnp.float32)
    pie = cs - 1.0 + carry_ref[...]
    keep = jnp.where(pie < CAP, sel, 0.0)
    kw_ref[...] = wfull * keep
    carry_ref[...] += cs[BM - 1:BM, :]


def _route(o_cat, w_o, hidden, post_ln_w, w_router, e_bias):
    return pl.pallas_call(
        _route_body,
        grid=(T // BM,),
        in_specs=[
            pl.BlockSpec((BM, H * DV), lambda i: (i, 0)),
            pl.BlockSpec((H * DV, D), lambda i: (0, 0)),
            pl.BlockSpec((BM, D), lambda i: (i, 0)),
            pl.BlockSpec((1, D), lambda i: (0, 0)),
            pl.BlockSpec((D, E), lambda i: (0, 0)),
            pl.BlockSpec((1, E), lambda i: (0, 0)),
        ],
        out_specs=[
            pl.BlockSpec((BM, D), lambda i: (i, 0)),
            pl.BlockSpec((BM, D), lambda i: (i, 0)),
            pl.BlockSpec((BM, E), lambda i: (i, 0)),
        ],
        out_shape=[
            jax.ShapeDtypeStruct((T, D), jnp.float32),
            jax.ShapeDtypeStruct((T, D), jnp.bfloat16),
            jax.ShapeDtypeStruct((T, E), jnp.float32),
        ],
        scratch_shapes=[pltpu.VMEM((1, E), jnp.float32)],
    )(o_cat, w_o, hidden, post_ln_w.reshape(1, D), w_router,
      e_bias.reshape(1, E))


EPP = 4


def _exp_body(x2_ref, wg_ref, wu_ref, kw_ref, wd_ref, hg_ref, wds_ref):
    pid = pl.program_id(0)
    wds_ref[...] = (wd_ref[...].reshape(EPP * F, D) * 32.0
                    ).astype(jnp.float8_e4m3fn)
    x2 = x2_ref[...].astype(jnp.float8_e4m3fn)
    wg = (jnp.concatenate([wg_ref[j] for j in range(EPP)], axis=1) * 32.0
          ).astype(jnp.float8_e4m3fn)
    wu = (jnp.concatenate([wu_ref[j] for j in range(EPP)], axis=1) * 32.0
          ).astype(jnp.float8_e4m3fn)
    xg = jnp.dot(x2, wg, preferred_element_type=jnp.float32) * (1.0 / 32.0)
    xu = jnp.dot(x2, wu, preferred_element_type=jnp.float32) * (1.0 / 32.0)
    iot = jax.lax.broadcasted_iota(jnp.int32, (1, E), 1)
    kw = kw_ref[...]
    scale = jnp.concatenate(
        [jnp.broadcast_to(
            jnp.sum(kw * (iot == pid * EPP + j).astype(jnp.float32),
                    axis=1, keepdims=True), (T, F))
         for j in range(EPP)], axis=1)
    hg_ref[...] = (jax.nn.silu(xg) * xu * scale * 8.0
                   ).astype(jnp.float8_e4m3fn)


def _experts_hg(x2b, w_g, w_u, keepw, w_d):
    return pl.pallas_call(
        _exp_body,
        grid=(E // EPP,),
        in_specs=[
            pl.BlockSpec((T, D), lambda e: (0, 0)),
            pl.BlockSpec((EPP, D, F), lambda e: (e, 0, 0)),
            pl.BlockSpec((EPP, D, F), lambda e: (e, 0, 0)),
            pl.BlockSpec((T, E), lambda e: (0, 0)),
            pl.BlockSpec((EPP, F, D), lambda e: (e, 0, 0)),
        ],
        out_specs=[
            pl.BlockSpec((T, EPP * F), lambda e: (0, e)),
            pl.BlockSpec((EPP * F, D), lambda e: (e, 0)),
        ],
        out_shape=[
            jax.ShapeDtypeStruct((T, E * F), jnp.float8_e4m3fn),
            jax.ShapeDtypeStruct((E * F, D), jnp.float8_e4m3fn),
        ],
    )(x2b, w_g, w_u, keepw, w_d)


def _final_body(hg_ref, wd_ref, h_ref, x2_ref, wsg_ref, wsu_ref, wsd_ref,
                o_ref):
    x2 = x2_ref[...]
    sg = jnp.dot(x2, wsg_ref[...].astype(jnp.bfloat16),
                 preferred_element_type=jnp.float32)
    su = jnp.dot(x2, wsu_ref[...].astype(jnp.bfloat16),
                 preferred_element_type=jnp.float32)
    hs = (jax.nn.silu(sg) * su).astype(jnp.bfloat16)
    shared = jnp.dot(hs, wsd_ref[...].astype(jnp.bfloat16),
                     preferred_element_type=jnp.float32)
    routed = jnp.dot(hg_ref[...], wd_ref[...],
                     preferred_element_type=jnp.float32) * (1.0 / 256.0)
    o_ref[...] = h_ref[...] + routed + shared


def _final(hg, wd_stack_bf16, h, x2b, ws_g, ws_u, ws_d):
    return pl.pallas_call(
        _final_body,
        grid=(T // BM,),
        in_specs=[
            pl.BlockSpec((BM, E * F), lambda i: (i, 0)),
            pl.BlockSpec((E * F, D), lambda i: (0, 0)),
            pl.BlockSpec((BM, D), lambda i: (i, 0)),
            pl.BlockSpec((BM, D), lambda i: (i, 0)),
            pl.BlockSpec((D, F), lambda i: (0, 0)),
            pl.BlockSpec((D, F), lambda i: (0, 0)),
            pl.BlockSpec((F, D), lambda i: (0, 0)),
        ],
        out_specs=pl.BlockSpec((BM, D), lambda i: (i, 0)),
        out_shape=jax.ShapeDtypeStruct((T, D), jnp.float32),
    )(hg, wd_stack_bf16, h, x2b, ws_g, ws_u, ws_d)


def kernel(hidden_states, positions, input_ln_w, post_ln_w, W_qkv_a,
           q_a_ln_w, W_q_b, kv_a_ln_w, W_kv_b, W_o, W_router, e_bias,
           W_g, W_u, W_d, Ws_g, Ws_u, Ws_d):
    inv = 1.0 / (10000.0 ** (jnp.arange(0, DR, 2, dtype=jnp.float32) / DR))
    ang = positions.astype(jnp.float32)[:, None] * inv[None, :]
    cos2 = jnp.repeat(jnp.cos(ang), 2, axis=1)
    sin2 = jnp.repeat(jnp.sin(ang), 2, axis=1)

    q, kf, v = _pre(hidden_states, input_ln_w, W_qkv_a, q_a_ln_w, W_q_b,
                    kv_a_ln_w, W_kv_b, cos2, sin2)

    o_cat = _attention(q, kf, v, cos2, sin2)
    h, x2b, keepw = _route(o_cat, W_o, hidden_states, post_ln_w,
                           W_router, e_bias)

    hg, wd_stack = _experts_hg(x2b, W_g, W_u, keepw, W_d)
    out = _final(hg, wd_stack, h, x2b, Ws_g, Ws_u, Ws_d)
    return out
